# baseline (speedup 1.0000x reference)
import functools

import jax
import jax.numpy as jnp
from jax import lax
from jax.experimental import pallas as pl
from jax.experimental.pallas import tpu as pltpu
from jax.experimental.pallas import tpu_sc as plsc

_EPS = 1e-08
_N_SPLITS = 8
_N_EXPERTS = 64
_LANES = 16


def _make_fused(v, d, b):
    info = plsc.get_sparse_core_info()
    nc, ns = info.num_cores, info.num_subcores
    d_half = d // nc
    rows_per_tile = d_half // ns
    tiles_per_split = _N_EXPERTS // rows_per_tile
    npart = 2
    win = 2048
    n_win = b // win
    groups = win // _LANES
    vvec = v // _LANES
    mesh = plsc.VectorSubcoreMesh(core_axis_name="c", subcore_axis_name="s")

    @functools.partial(
        pl.kernel,
        out_type=jax.ShapeDtypeStruct((d, b), jnp.float32),
        mesh=mesh,
        scratch_types=[
            pltpu.VMEM((rows_per_tile * v,), jnp.float32),
            pltpu.VMEM((tiles_per_split, npart, v), jnp.float32),
            pltpu.VMEM_SHARED((ns, npart, v), jnp.float32),
            pltpu.VMEM((2, win), jnp.int32),
            pltpu.VMEM((2, rows_per_tile, win), jnp.float32),
        ]
        + [pltpu.SemaphoreType.DMA] * 3,
        compiler_params=pltpu.CompilerParams(needs_layout_passes=False),
    )
    def fused(logits_t_hbm, tasks_hbm, out_hbm, blk, part_rd, parts_sh, tw, os_, *sems):
        tsem, s0, s1 = sems
        ssems = (s0, s1)
        cid = lax.axis_index("c")
        sid = lax.axis_index("s")

        tw_h = pltpu.async_copy(tasks_hbm.at[pl.ds(0, win)], tw.at[0], tsem)

        j0 = pl.multiple_of(cid * d_half + sid * rows_per_tile, rows_per_tile)
        pltpu.sync_copy(
            logits_t_hbm.at[pl.ds(j0 * v, rows_per_tile * v)], blk
        )

        def sig_vec(q, carry):
            x = blk[pl.ds(q * _LANES, _LANES)]
            blk[pl.ds(q * _LANES, _LANES)] = 1.0 / (1.0 + jnp.exp(-x))
            return carry

        lax.fori_loop(0, rows_per_tile * vvec, sig_vec, 0)

        half_rows = rows_per_tile // npart

        def part_vec(q, carry):
            for h in range(npart):
                acc = blk[pl.ds(h * half_rows * v + q * _LANES, _LANES)]
                for r in range(1, half_rows):
                    acc = acc + blk[
                        pl.ds((h * half_rows + r) * v + q * _LANES, _LANES)
                    ]
                part_rd[0, h, pl.ds(q * _LANES, _LANES)] = acc
            return carry

        lax.fori_loop(0, vvec, part_vec, 0)
        pltpu.sync_copy(part_rd.at[0], parts_sh.at[sid])
        plsc.subcore_barrier()

        g0 = (sid // tiles_per_split) * tiles_per_split
        pltpu.sync_copy(parts_sh.at[pl.ds(g0, tiles_per_split)], part_rd)

        def norm_vec(q, carry):
            acc = part_rd[0, 0, pl.ds(q * _LANES, _LANES)]
            for t in range(tiles_per_split):
                for h in range(npart):
                    if t == 0 and h == 0:
                        continue
                    acc = acc + part_rd[t, h, pl.ds(q * _LANES, _LANES)]
            denom = acc + _EPS
            for r in range(rows_per_tile):
                blk[pl.ds(r * v + q * _LANES, _LANES)] = (
                    blk[pl.ds(r * v + q * _LANES, _LANES)] / denom
                )
            return carry

        lax.fori_loop(0, vvec, norm_vec, 0)


        def do_window(w, wb, handle):
            handle.wait()
            nxt = w + 1
            nh = handle
            if nxt < n_win:
                nh = pltpu.async_copy(
                    tasks_hbm.at[pl.ds(nxt * win, win)], tw.at[1 - wb], tsem
                )

            def gather_group(g, carry):
                tv = tw[wb, pl.ds(g * _LANES, _LANES)]
                for jp in range(rows_per_tile):
                    os_[wb, jp, pl.ds(g * _LANES, _LANES)] = plsc.load_gather(
                        blk, [tv + jp * v]
                    )
                return carry

            lax.fori_loop(0, groups, gather_group, 0)
            sh = pltpu.async_copy(
                os_.at[wb],
                out_hbm.at[pl.ds(j0, rows_per_tile), pl.ds(w * win, win)],
                ssems[wb],
            )
            return nh, sh

        pending = [None, None]
        h = tw_h
        for w in range(n_win):
            wb = w % 2
            if pending[wb] is not None:
                pending[wb].wait()
            h, pending[wb] = do_window(w, wb, h)
        for p in pending:
            if p is not None:
                p.wait()

    return fused


def kernel(x, task_ids, module_logits):
    n_tokens = task_ids.shape[0]
    v, d = module_logits.shape
    idx = task_ids.astype(jnp.int32)
    logits_t = jnp.transpose(module_logits, (1, 0)).reshape(-1)
    out_t = _make_fused(v, d, n_tokens)(logits_t, idx)
    return jnp.transpose(out_t, (1, 0)).reshape(n_tokens, _N_SPLITS, _N_EXPERTS)


# speedup vs baseline: 2.1457x; 2.1457x over previous
import functools

import jax
import jax.numpy as jnp
from jax import lax
from jax.experimental import pallas as pl
from jax.experimental.pallas import tpu as pltpu
from jax.experimental.pallas import tpu_sc as plsc

_EPS = 1e-08
_N_SPLITS = 8
_N_EXPERTS = 64
_LANES = 16


def _make_fused(v, d, b):
    info = plsc.get_sparse_core_info()
    nc, ns = info.num_cores, info.num_subcores
    d_half = d // nc
    rows_per_tile = d_half // ns
    tiles_per_split = _N_EXPERTS // rows_per_tile
    npart = 2
    win = 2048
    n_win = b // win
    groups = win // _LANES
    vvec = v // _LANES
    mesh = plsc.VectorSubcoreMesh(core_axis_name="c", subcore_axis_name="s")

    @functools.partial(
        pl.kernel,
        out_type=jax.ShapeDtypeStruct((d, b), jnp.float32),
        mesh=mesh,
        scratch_types=[
            pltpu.VMEM((rows_per_tile * v,), jnp.float32),
            pltpu.VMEM((tiles_per_split, npart, v), jnp.float32),
            pltpu.VMEM_SHARED((ns, npart, v), jnp.float32),
            pltpu.VMEM((2, win), jnp.int32),
            pltpu.VMEM((2, rows_per_tile, win), jnp.float32),
        ]
        + [pltpu.SemaphoreType.DMA] * 3,
        compiler_params=pltpu.CompilerParams(needs_layout_passes=False),
    )
    def fused(logits_t_hbm, tasks_hbm, out_hbm, blk, part_rd, parts_sh, tw, os_, *sems):
        tsem, s0, s1 = sems
        ssems = (s0, s1)
        cid = lax.axis_index("c")
        sid = lax.axis_index("s")

        tw_h = pltpu.async_copy(tasks_hbm.at[pl.ds(0, win)], tw.at[0], tsem)

        j0 = pl.multiple_of(cid * d_half + sid * rows_per_tile, rows_per_tile)
        pltpu.sync_copy(
            logits_t_hbm.at[pl.ds(j0 * v, rows_per_tile * v)], blk
        )

        @plsc.parallel_loop(0, rows_per_tile * vvec, unroll=4)
        def sig_vec(q):
            x = blk[pl.ds(q * _LANES, _LANES)]
            blk[pl.ds(q * _LANES, _LANES)] = 1.0 / (1.0 + jnp.exp(-x))

        half_rows = rows_per_tile // npart

        @plsc.parallel_loop(0, vvec, unroll=2)
        def part_vec(q):
            for h in range(npart):
                acc = blk[pl.ds(h * half_rows * v + q * _LANES, _LANES)]
                for r in range(1, half_rows):
                    acc = acc + blk[
                        pl.ds((h * half_rows + r) * v + q * _LANES, _LANES)
                    ]
                part_rd[0, h, pl.ds(q * _LANES, _LANES)] = acc
        pltpu.sync_copy(part_rd.at[0], parts_sh.at[sid])
        plsc.subcore_barrier()

        g0 = (sid // tiles_per_split) * tiles_per_split
        pltpu.sync_copy(parts_sh.at[pl.ds(g0, tiles_per_split)], part_rd)

        @plsc.parallel_loop(0, vvec, unroll=2)
        def norm_vec(q):
            acc = part_rd[0, 0, pl.ds(q * _LANES, _LANES)]
            for t in range(tiles_per_split):
                for h in range(npart):
                    if t == 0 and h == 0:
                        continue
                    acc = acc + part_rd[t, h, pl.ds(q * _LANES, _LANES)]
            denom = acc + _EPS
            for r in range(rows_per_tile):
                blk[pl.ds(r * v + q * _LANES, _LANES)] = (
                    blk[pl.ds(r * v + q * _LANES, _LANES)] / denom
                )


        def do_window(w, wb, handle):
            handle.wait()
            nxt = w + 1
            nh = handle
            if nxt < n_win:
                nh = pltpu.async_copy(
                    tasks_hbm.at[pl.ds(nxt * win, win)], tw.at[1 - wb], tsem
                )

            @plsc.parallel_loop(0, groups, unroll=4)
            def gather_group(g):
                tv = tw[wb, pl.ds(g * _LANES, _LANES)]
                for jp in range(rows_per_tile):
                    os_[wb, jp, pl.ds(g * _LANES, _LANES)] = plsc.load_gather(
                        blk, [tv + jp * v]
                    )
            sh = pltpu.async_copy(
                os_.at[wb],
                out_hbm.at[pl.ds(j0, rows_per_tile), pl.ds(w * win, win)],
                ssems[wb],
            )
            return nh, sh

        pending = [None, None]
        h = tw_h
        for w in range(n_win):
            wb = w % 2
            if pending[wb] is not None:
                pending[wb].wait()
            h, pending[wb] = do_window(w, wb, h)
        for p in pending:
            if p is not None:
                p.wait()

    return fused


def kernel(x, task_ids, module_logits):
    n_tokens = task_ids.shape[0]
    v, d = module_logits.shape
    idx = task_ids.astype(jnp.int32)
    logits_t = jnp.transpose(module_logits, (1, 0)).reshape(-1)
    out_t = _make_fused(v, d, n_tokens)(logits_t, idx)
    return jnp.transpose(out_t, (1, 0)).reshape(n_tokens, _N_SPLITS, _N_EXPERTS)
